# baseline (speedup 1.0000x reference)
import functools

import jax
import jax.numpy as jnp
from jax import lax
from jax.experimental import pallas as pl
from jax.experimental.pallas import tpu as pltpu
from jax.experimental.pallas import tpu_sc as plsc

BATCH = 16384
D = 64
DP = 2 * D
NC, NS = 2, 16
NW = NC * NS
BPW = BATCH // NW
CHUNK = 128
NCHUNK = BPW // CHUNK


def _tc_pack(table, blk2):
    n2 = table.shape[0] // 2
    nblk = n2 // blk2

    def body(a_ref, b_ref, o_ref):
        o_ref[:, :D] = a_ref[...]
        o_ref[:, D:] = b_ref[...]

    return pl.pallas_call(
        body,
        out_shape=jax.ShapeDtypeStruct((n2, DP), jnp.float32),
        grid=(nblk,),
        in_specs=[
            pl.BlockSpec((blk2, D), lambda i: (i, 0)),
            pl.BlockSpec((blk2, D), lambda i, _n=nblk: (i + _n, 0)),
        ],
        out_specs=pl.BlockSpec((blk2, DP), lambda i: (i, 0)),
    )(table, table)


def _tc_pack_small(table):
    n2 = table.shape[0] // 2

    def body(x_ref, o_ref):
        x = x_ref[...]
        o_ref[...] = jnp.concatenate([x[:n2], x[n2:]], axis=1)

    return pl.pallas_call(
        body,
        out_shape=jax.ShapeDtypeStruct((n2, DP), jnp.float32),
    )(table)


def _sc_gather_all(heads, rels, tails, enth_p, entt_p, rel_p, relinv_p):
    mesh = plsc.VectorSubcoreMesh(core_axis_name="c", subcore_axis_name="s")
    row_ty = jax.ShapeDtypeStruct((BATCH, DP), jnp.float32)

    @functools.partial(
        pl.kernel,
        out_type=(row_ty,) * 6,
        mesh=mesh,
        scratch_types=[
            pltpu.VMEM((BPW,), jnp.int32),
            pltpu.VMEM((BPW,), jnp.int32),
            pltpu.VMEM((BPW,), jnp.int32),
        ] + [pltpu.VMEM((CHUNK, DP), jnp.float32)] * 6 + [
            pltpu.SemaphoreType.DMA,
        ],
    )
    def k(heads_hbm, rels_hbm, tails_hbm, enth_hbm, entt_hbm, rel_hbm,
          relinv_hbm, hh_out, ht_out, th_out, tt_out, r_out, rinv_out,
          hv, rv, tv, b0, b1, b2, b3, b4, b5, sem):
        wid = lax.axis_index("s") * NC + lax.axis_index("c")
        base = wid * BPW
        pltpu.sync_copy(heads_hbm.at[pl.ds(base, BPW)], hv)
        pltpu.sync_copy(rels_hbm.at[pl.ds(base, BPW)], rv)
        pltpu.sync_copy(tails_hbm.at[pl.ds(base, BPW)], tv)
        for iv, half in ((hv, enth_hbm.shape[0]), (rv, rel_hbm.shape[0]),
                         (tv, entt_hbm.shape[0])):
            @pl.loop(0, BPW, step=16)
            def _(i, _iv=iv, _half=half):
                v = _iv[pl.ds(i, 16)]
                _iv[pl.ds(i, 16)] = jnp.where(
                    v >= _half, v - _half, v)

        bufs = (b0, b1, b2, b3, b4, b5)
        outs = (hh_out, ht_out, th_out, tt_out, r_out, rinv_out)
        for c in range(NCHUNK):
            cbase = c * CHUNK
            hidx = hv.at[pl.ds(cbase, CHUNK)]
            tidx = tv.at[pl.ds(cbase, CHUNK)]
            ridx = rv.at[pl.ds(cbase, CHUNK)]
            copies = [
                pltpu.async_copy(enth_hbm.at[hidx], b0, sem),
                pltpu.async_copy(enth_hbm.at[tidx], b1, sem),
                pltpu.async_copy(entt_hbm.at[hidx], b2, sem),
                pltpu.async_copy(entt_hbm.at[tidx], b3, sem),
                pltpu.async_copy(rel_hbm.at[ridx], b4, sem),
                pltpu.async_copy(relinv_hbm.at[ridx], b5, sem),
            ]
            for cp in copies:
                cp.wait()
            for buf, out in zip(bufs, outs):
                pltpu.sync_copy(buf, out.at[pl.ds(base + cbase, CHUNK)])

    return k(heads, rels, tails, enth_p, entt_p, rel_p, relinv_p)


def _tc_score(hh, ht, th, tt, r, rinv, ph, pt, pr):
    blk = 2048

    def sel(x_ref, p):
        lo = x_ref[:, :D]
        hi = x_ref[:, D:]
        return lo + p * (hi - lo)

    def body(hh_ref, ht_ref, th_ref, tt_ref, r_ref, rinv_ref,
             ph_ref, pt_ref, pr_ref, o_ref):
        p_h = ph_ref[...][:, None]
        p_t = pt_ref[...][:, None]
        p_r = pr_ref[...][:, None]
        hh_v = sel(hh_ref, p_h)
        ht_v = sel(ht_ref, p_t)
        th_v = sel(th_ref, p_h)
        tt_v = sel(tt_ref, p_t)
        r_v = sel(r_ref, p_r)
        rinv_v = sel(rinv_ref, p_r)
        f = jnp.sum(hh_v * r_v * tt_v, axis=1)
        inv = jnp.sum(ht_v * rinv_v * th_v, axis=1)
        o_ref[...] = jnp.clip((f + inv) * 0.5, -20.0, 20.0)

    return pl.pallas_call(
        body,
        out_shape=jax.ShapeDtypeStruct((BATCH,), jnp.float32),
        grid=(BATCH // blk,),
        in_specs=[pl.BlockSpec((blk, DP), lambda i: (i, 0))] * 6
        + [pl.BlockSpec((blk,), lambda i: (i,))] * 3,
        out_specs=pl.BlockSpec((blk,), lambda i: (i,)),
    )(hh, ht, th, tt, r, rinv, ph, pt, pr)


def kernel(heads, rels, tails, ent_h_embs, ent_t_embs, rel_embs, rel_inv_embs):
    heads = heads.astype(jnp.int32)
    rels = rels.astype(jnp.int32)
    tails = tails.astype(jnp.int32)
    enth_p = _tc_pack(ent_h_embs, 2000)
    entt_p = _tc_pack(ent_t_embs, 2000)
    rel_p = _tc_pack_small(rel_embs)
    relinv_p = _tc_pack_small(rel_inv_embs)
    hh, ht, th, tt, r, rinv = _sc_gather_all(
        heads, rels, tails, enth_p, entt_p, rel_p, relinv_p)
    n_ent_half = ent_h_embs.shape[0] // 2
    n_rel_half = rel_embs.shape[0] // 2
    ph = (heads >= n_ent_half).astype(jnp.float32)
    pt = (tails >= n_ent_half).astype(jnp.float32)
    pr = (rels >= n_rel_half).astype(jnp.float32)
    return _tc_score(hh, ht, th, tt, r, rinv, ph, pt, pr)


# speedup vs baseline: 1.1163x; 1.1163x over previous
import functools

import jax
import jax.numpy as jnp
from jax import lax
from jax.experimental import pallas as pl
from jax.experimental.pallas import tpu as pltpu
from jax.experimental.pallas import tpu_sc as plsc

BATCH = 16384
D = 64
DP = 2 * D
NC, NS = 2, 16
NW = NC * NS
BPW = BATCH // NW
CHUNK = 128
NCHUNK = BPW // CHUNK


def _tc_pack(table, blk2):
    n2 = table.shape[0] // 2
    nblk = n2 // blk2

    def body(a_ref, b_ref, o_ref):
        o_ref[:, :D] = a_ref[...]
        o_ref[:, D:] = b_ref[...]

    return pl.pallas_call(
        body,
        out_shape=jax.ShapeDtypeStruct((n2, DP), jnp.float32),
        grid=(nblk,),
        in_specs=[
            pl.BlockSpec((blk2, D), lambda i: (i, 0)),
            pl.BlockSpec((blk2, D), lambda i, _n=nblk: (i + _n, 0)),
        ],
        out_specs=pl.BlockSpec((blk2, DP), lambda i: (i, 0)),
    )(table, table)


def _tc_pack_small(table):
    n2 = table.shape[0] // 2

    def body(x_ref, o_ref):
        x = x_ref[...]
        o_ref[...] = jnp.concatenate([x[:n2], x[n2:]], axis=1)

    return pl.pallas_call(
        body,
        out_shape=jax.ShapeDtypeStruct((n2, DP), jnp.float32),
    )(table)


def _sc_gather_all(heads, rels, tails, enth_p, entt_p, rel_p, relinv_p):
    mesh = plsc.VectorSubcoreMesh(core_axis_name="c", subcore_axis_name="s")
    row_ty = jax.ShapeDtypeStruct((BATCH, DP), jnp.float32)

    @functools.partial(
        pl.kernel,
        out_type=(row_ty,) * 6,
        mesh=mesh,
        scratch_types=[
            pltpu.VMEM((BPW,), jnp.int32),
            pltpu.VMEM((BPW,), jnp.int32),
            pltpu.VMEM((BPW,), jnp.int32),
        ] + [pltpu.VMEM((CHUNK, DP), jnp.float32)] * 6 + [
            pltpu.SemaphoreType.DMA,
        ],
    )
    def k(heads_hbm, rels_hbm, tails_hbm, enth_hbm, entt_hbm, rel_hbm,
          relinv_hbm, hh_out, ht_out, th_out, tt_out, r_out, rinv_out,
          hv, rv, tv, b0, b1, b2, b3, b4, b5, sem):
        wid = lax.axis_index("s") * NC + lax.axis_index("c")
        base = wid * BPW
        pltpu.sync_copy(heads_hbm.at[pl.ds(base, BPW)], hv)
        pltpu.sync_copy(rels_hbm.at[pl.ds(base, BPW)], rv)
        pltpu.sync_copy(tails_hbm.at[pl.ds(base, BPW)], tv)
        for iv, half in ((hv, enth_hbm.shape[0]), (rv, rel_hbm.shape[0]),
                         (tv, entt_hbm.shape[0])):
            @pl.loop(0, BPW, step=16)
            def _(i, _iv=iv, _half=half):
                v = _iv[pl.ds(i, 16)]
                _iv[pl.ds(i, 16)] = jnp.where(
                    v >= _half, v - _half, v)

        bufs = (b0, b1, b2, b3, b4, b5)
        outs = (hh_out, ht_out, th_out, tt_out, r_out, rinv_out)
        for c in range(NCHUNK):
            cbase = c * CHUNK
            hidx = hv.at[pl.ds(cbase, CHUNK)]
            tidx = tv.at[pl.ds(cbase, CHUNK)]
            ridx = rv.at[pl.ds(cbase, CHUNK)]
            copies = [
                pltpu.async_copy(enth_hbm.at[hidx], b0, sem),
                pltpu.async_copy(enth_hbm.at[tidx], b1, sem),
                pltpu.async_copy(entt_hbm.at[hidx], b2, sem),
                pltpu.async_copy(entt_hbm.at[tidx], b3, sem),
                pltpu.async_copy(rel_hbm.at[ridx], b4, sem),
                pltpu.async_copy(relinv_hbm.at[ridx], b5, sem),
            ]
            for cp in copies:
                cp.wait()
            for buf, out in zip(bufs, outs):
                pltpu.sync_copy(buf, out.at[pl.ds(base + cbase, CHUNK)])

    return k(heads, rels, tails, enth_p, entt_p, rel_p, relinv_p)


def _tc_score(hh, ht, th, tt, r, rinv, ph, pt, pr):
    blk = 2048

    def sel(x_ref, p):
        lo = x_ref[:, :D]
        hi = x_ref[:, D:]
        return lo + p * (hi - lo)

    def body(hh_ref, ht_ref, th_ref, tt_ref, r_ref, rinv_ref,
             ph_ref, pt_ref, pr_ref, o_ref):
        p_h = ph_ref[...][:, None]
        p_t = pt_ref[...][:, None]
        p_r = pr_ref[...][:, None]
        hh_v = sel(hh_ref, p_h)
        ht_v = sel(ht_ref, p_t)
        th_v = sel(th_ref, p_h)
        tt_v = sel(tt_ref, p_t)
        r_v = sel(r_ref, p_r)
        rinv_v = sel(rinv_ref, p_r)
        f = jnp.sum(hh_v * r_v * tt_v, axis=1)
        inv = jnp.sum(ht_v * rinv_v * th_v, axis=1)
        o_ref[...] = jnp.clip((f + inv) * 0.5, -20.0, 20.0)

    return pl.pallas_call(
        body,
        out_shape=jax.ShapeDtypeStruct((BATCH,), jnp.float32),
        grid=(BATCH // blk,),
        in_specs=[pl.BlockSpec((blk, DP), lambda i: (i, 0))] * 6
        + [pl.BlockSpec((blk,), lambda i: (i,))] * 3,
        out_specs=pl.BlockSpec((blk,), lambda i: (i,)),
    )(hh, ht, th, tt, r, rinv, ph, pt, pr)


def kernel(heads, rels, tails, ent_h_embs, ent_t_embs, rel_embs, rel_inv_embs):
    heads = heads.astype(jnp.int32)
    rels = rels.astype(jnp.int32)
    tails = tails.astype(jnp.int32)
    enth_p = _tc_pack(ent_h_embs, 10000)
    entt_p = _tc_pack(ent_t_embs, 10000)
    rel_p = _tc_pack_small(rel_embs)
    relinv_p = _tc_pack_small(rel_inv_embs)
    hh, ht, th, tt, r, rinv = _sc_gather_all(
        heads, rels, tails, enth_p, entt_p, rel_p, relinv_p)
    n_ent_half = ent_h_embs.shape[0] // 2
    n_rel_half = rel_embs.shape[0] // 2
    ph = (heads >= n_ent_half).astype(jnp.float32)
    pt = (tails >= n_ent_half).astype(jnp.float32)
    pr = (rels >= n_rel_half).astype(jnp.float32)
    return _tc_score(hh, ht, th, tt, r, rinv, ph, pt, pr)


# speedup vs baseline: 1.1785x; 1.0557x over previous
import functools

import jax
import jax.numpy as jnp
from jax import lax
from jax.experimental import pallas as pl
from jax.experimental.pallas import tpu as pltpu
from jax.experimental.pallas import tpu_sc as plsc

BATCH = 16384
D = 64
DP = 2 * D
NC, NS = 2, 16
NW = NC * NS
BPW = BATCH // NW
CHUNK = 128
NCHUNK = BPW // CHUNK


def _tc_pack(table, blk2):
    n2 = table.shape[0] // 2
    nblk = n2 // blk2

    def body(a_ref, b_ref, o_ref):
        o_ref[:, :D] = a_ref[...]
        o_ref[:, D:] = b_ref[...]

    return pl.pallas_call(
        body,
        out_shape=jax.ShapeDtypeStruct((n2, DP), jnp.float32),
        grid=(nblk,),
        in_specs=[
            pl.BlockSpec((blk2, D), lambda i: (i, 0)),
            pl.BlockSpec((blk2, D), lambda i, _n=nblk: (i + _n, 0)),
        ],
        out_specs=pl.BlockSpec((blk2, DP), lambda i: (i, 0)),
    )(table, table)


def _tc_pack_small(table):
    n2 = table.shape[0] // 2

    def body(x_ref, o_ref):
        x = x_ref[...]
        o_ref[...] = jnp.concatenate([x[:n2], x[n2:]], axis=1)

    return pl.pallas_call(
        body,
        out_shape=jax.ShapeDtypeStruct((n2, DP), jnp.float32),
    )(table)


def _sc_gather_all(heads, rels, tails, enth_p, entt_p, rel_p, relinv_p):
    mesh = plsc.VectorSubcoreMesh(core_axis_name="c", subcore_axis_name="s")
    row_ty = jax.ShapeDtypeStruct((BATCH, DP), jnp.float32)

    @functools.partial(
        pl.kernel,
        out_type=(row_ty,) * 6,
        mesh=mesh,
        scratch_types=[
            pltpu.VMEM((BPW,), jnp.int32),
            pltpu.VMEM((BPW,), jnp.int32),
            pltpu.VMEM((BPW,), jnp.int32),
        ] + [pltpu.VMEM((CHUNK, DP), jnp.float32)] * 6 + [
            pltpu.SemaphoreType.DMA,
        ],
    )
    def k(heads_hbm, rels_hbm, tails_hbm, enth_hbm, entt_hbm, rel_hbm,
          relinv_hbm, hh_out, ht_out, th_out, tt_out, r_out, rinv_out,
          hv, rv, tv, b0, b1, b2, b3, b4, b5, sem):
        wid = lax.axis_index("s") * NC + lax.axis_index("c")
        base = wid * BPW
        pltpu.sync_copy(heads_hbm.at[pl.ds(base, BPW)], hv)
        pltpu.sync_copy(rels_hbm.at[pl.ds(base, BPW)], rv)
        pltpu.sync_copy(tails_hbm.at[pl.ds(base, BPW)], tv)
        for iv in (hv, rv, tv):
            @pl.loop(0, BPW, step=16)
            def _(i, _iv=iv):
                _iv[pl.ds(i, 16)] = lax.shift_right_logical(
                    _iv[pl.ds(i, 16)], 1)

        bufs = (b0, b1, b2, b3, b4, b5)
        outs = (hh_out, ht_out, th_out, tt_out, r_out, rinv_out)
        for c in range(NCHUNK):
            cbase = c * CHUNK
            hidx = hv.at[pl.ds(cbase, CHUNK)]
            tidx = tv.at[pl.ds(cbase, CHUNK)]
            ridx = rv.at[pl.ds(cbase, CHUNK)]
            copies = [
                pltpu.async_copy(enth_hbm.at[hidx], b0, sem),
                pltpu.async_copy(enth_hbm.at[tidx], b1, sem),
                pltpu.async_copy(entt_hbm.at[hidx], b2, sem),
                pltpu.async_copy(entt_hbm.at[tidx], b3, sem),
                pltpu.async_copy(rel_hbm.at[ridx], b4, sem),
                pltpu.async_copy(relinv_hbm.at[ridx], b5, sem),
            ]
            for cp in copies:
                cp.wait()
            for buf, out in zip(bufs, outs):
                pltpu.sync_copy(buf, out.at[pl.ds(base + cbase, CHUNK)])

    return k(heads, rels, tails, enth_p, entt_p, rel_p, relinv_p)


def _tc_score(hh, ht, th, tt, r, rinv, ph, pt, pr):
    blk = 2048

    def sel(x_ref, p):
        lo = x_ref[:, :D]
        hi = x_ref[:, D:]
        return lo + p * (hi - lo)

    def body(hh_ref, ht_ref, th_ref, tt_ref, r_ref, rinv_ref,
             ph_ref, pt_ref, pr_ref, o_ref):
        p_h = ph_ref[...][:, None]
        p_t = pt_ref[...][:, None]
        p_r = pr_ref[...][:, None]
        hh_v = sel(hh_ref, p_h)
        ht_v = sel(ht_ref, p_t)
        th_v = sel(th_ref, p_h)
        tt_v = sel(tt_ref, p_t)
        r_v = sel(r_ref, p_r)
        rinv_v = sel(rinv_ref, p_r)
        f = jnp.sum(hh_v * r_v * tt_v, axis=1)
        inv = jnp.sum(ht_v * rinv_v * th_v, axis=1)
        o_ref[...] = jnp.clip((f + inv) * 0.5, -20.0, 20.0)

    return pl.pallas_call(
        body,
        out_shape=jax.ShapeDtypeStruct((BATCH,), jnp.float32),
        grid=(BATCH // blk,),
        in_specs=[pl.BlockSpec((blk, DP), lambda i: (i, 0))] * 6
        + [pl.BlockSpec((blk,), lambda i: (i,))] * 3,
        out_specs=pl.BlockSpec((blk,), lambda i: (i,)),
    )(hh, ht, th, tt, r, rinv, ph, pt, pr)


def kernel(heads, rels, tails, ent_h_embs, ent_t_embs, rel_embs, rel_inv_embs):
    heads = heads.astype(jnp.int32)
    rels = rels.astype(jnp.int32)
    tails = tails.astype(jnp.int32)
    enth_p = ent_h_embs.reshape(-1, DP)
    entt_p = ent_t_embs.reshape(-1, DP)
    rel_p = rel_embs.reshape(-1, DP)
    relinv_p = rel_inv_embs.reshape(-1, DP)
    hh, ht, th, tt, r, rinv = _sc_gather_all(
        heads, rels, tails, enth_p, entt_p, rel_p, relinv_p)
    ph = (heads & 1).astype(jnp.float32)
    pt = (tails & 1).astype(jnp.float32)
    pr = (rels & 1).astype(jnp.float32)
    return _tc_score(hh, ht, th, tt, r, rinv, ph, pt, pr)


# speedup vs baseline: 1.8372x; 1.5590x over previous
import functools

import jax
import jax.numpy as jnp
from jax import lax
from jax.experimental import pallas as pl
from jax.experimental.pallas import tpu as pltpu
from jax.experimental.pallas import tpu_sc as plsc

BATCH = 16384
D = 64
DP = 2 * D
NC, NS = 2, 16
NW = NC * NS
BPW = BATCH // NW
CHUNK = 128
NCHUNK = BPW // CHUNK


def _sc_gather_all(heads, rels, tails, ent_h, ent_t, rel, rel_inv):
    mesh = plsc.VectorSubcoreMesh(core_axis_name="c", subcore_axis_name="s")
    row_ty = jax.ShapeDtypeStruct((BATCH, DP), jnp.float32)

    @functools.partial(
        pl.kernel,
        out_type=(row_ty,) * 6,
        mesh=mesh,
        scratch_types=[
            pltpu.VMEM((BPW,), jnp.int32),
            pltpu.VMEM((BPW,), jnp.int32),
            pltpu.VMEM((BPW,), jnp.int32),
        ] + [pltpu.VMEM((CHUNK, D), jnp.float32)] * 6 + [
            pltpu.VMEM((CHUNK, DP), jnp.float32),
            pltpu.SemaphoreType.DMA,
        ],
    )
    def k(heads_hbm, rels_hbm, tails_hbm, enth_hbm, entt_hbm, rel_hbm,
          relinv_hbm, hh_out, ht_out, th_out, tt_out, r_out, rinv_out,
          hv, rv, tv, b0, b1, b2, b3, b4, b5, pk, sem):
        wid = lax.axis_index("s") * NC + lax.axis_index("c")
        base = wid * BPW
        pltpu.sync_copy(heads_hbm.at[pl.ds(base, BPW)], hv)
        pltpu.sync_copy(rels_hbm.at[pl.ds(base, BPW)], rv)
        pltpu.sync_copy(tails_hbm.at[pl.ds(base, BPW)], tv)
        bufs = (b0, b1, b2, b3, b4, b5)
        outs = (hh_out, ht_out, th_out, tt_out, r_out, rinv_out)
        for c in range(NCHUNK):
            cbase = c * CHUNK

            @pl.loop(0, CHUNK, step=16)
            def _(i):
                hvec = hv[pl.ds(cbase + i, 16)]
                tvec = tv[pl.ds(cbase + i, 16)]
                rvec = rv[pl.ds(cbase + i, 16)]
                for j in range(16):
                    h = hvec[j]
                    t = tvec[j]
                    r = rvec[j]
                    dst = pl.ds(i + j, 1)
                    pltpu.async_copy(
                        enth_hbm.at[pl.ds(h, 1)], b0.at[dst], sem)
                    pltpu.async_copy(
                        enth_hbm.at[pl.ds(t, 1)], b1.at[dst], sem)
                    pltpu.async_copy(
                        entt_hbm.at[pl.ds(h, 1)], b2.at[dst], sem)
                    pltpu.async_copy(
                        entt_hbm.at[pl.ds(t, 1)], b3.at[dst], sem)
                    pltpu.async_copy(
                        rel_hbm.at[pl.ds(r, 1)], b4.at[dst], sem)
                    pltpu.async_copy(
                        relinv_hbm.at[pl.ds(r, 1)], b5.at[dst], sem)

            for buf in bufs:
                pltpu.make_async_copy(
                    enth_hbm.at[pl.ds(0, CHUNK)], buf, sem).wait()
            for buf, out in zip(bufs, outs):
                @pl.loop(0, CHUNK)
                def _(q, _buf=buf):
                    for kk in range(D // 16):
                        slc = (pl.ds(q, 1), pl.ds(kk * 16, 16))
                        pk.at[slc][...] = _buf.at[slc][...]

                pltpu.sync_copy(pk, out.at[pl.ds(base + cbase, CHUNK)])

    return k(heads, rels, tails, ent_h, ent_t, rel, rel_inv)


def _tc_score(hh, ht, th, tt, r, rinv):
    blk = 2048

    def body(hh_ref, ht_ref, th_ref, tt_ref, r_ref, rinv_ref, o_ref):
        s = (slice(None), slice(0, D))
        f = jnp.sum(hh_ref[s] * r_ref[s] * tt_ref[s], axis=1)
        inv = jnp.sum(ht_ref[s] * rinv_ref[s] * th_ref[s], axis=1)
        o_ref[...] = jnp.clip((f + inv) * 0.5, -20.0, 20.0)

    return pl.pallas_call(
        body,
        out_shape=jax.ShapeDtypeStruct((BATCH,), jnp.float32),
        grid=(BATCH // blk,),
        in_specs=[pl.BlockSpec((blk, DP), lambda i: (i, 0))] * 6,
        out_specs=pl.BlockSpec((blk,), lambda i: (i,)),
    )(hh, ht, th, tt, r, rinv)


def kernel(heads, rels, tails, ent_h_embs, ent_t_embs, rel_embs, rel_inv_embs):
    heads = heads.astype(jnp.int32)
    rels = rels.astype(jnp.int32)
    tails = tails.astype(jnp.int32)
    hh, ht, th, tt, r, rinv = _sc_gather_all(
        heads, rels, tails, ent_h_embs, ent_t_embs, rel_embs, rel_inv_embs)
    return _tc_score(hh, ht, th, tt, r, rinv)


# speedup vs baseline: 1.8390x; 1.0010x over previous
import functools

import jax
import jax.numpy as jnp
from jax import lax
from jax.experimental import pallas as pl
from jax.experimental.pallas import tpu as pltpu
from jax.experimental.pallas import tpu_sc as plsc

BATCH = 16384
D = 64
DP = 2 * D
NC, NS = 2, 16
NW = NC * NS
BPW = BATCH // NW
CHUNK = 128
NCHUNK = BPW // CHUNK


def _sc_gather_all(heads, rels, tails, ent_h, ent_t, rel, rel_inv):
    mesh = plsc.VectorSubcoreMesh(core_axis_name="c", subcore_axis_name="s")
    row_ty = jax.ShapeDtypeStruct((BATCH, DP), jnp.float32)

    @functools.partial(
        pl.kernel,
        out_type=(row_ty,) * 6,
        mesh=mesh,
        scratch_types=[
            pltpu.VMEM((BPW + 16,), jnp.int32),
            pltpu.VMEM((BPW + 16,), jnp.int32),
            pltpu.VMEM((BPW + 16,), jnp.int32),
        ] + [pltpu.VMEM((CHUNK, D), jnp.float32)] * 6 + [
            pltpu.VMEM((CHUNK, DP), jnp.float32),
            pltpu.SemaphoreType.DMA,
        ],
    )
    def k(heads_hbm, rels_hbm, tails_hbm, enth_hbm, entt_hbm, rel_hbm,
          relinv_hbm, hh_out, ht_out, th_out, tt_out, r_out, rinv_out,
          hv, rv, tv, b0, b1, b2, b3, b4, b5, pk, sem):
        wid = lax.axis_index("s") * NC + lax.axis_index("c")
        base = wid * BPW
        pltpu.sync_copy(heads_hbm.at[pl.ds(base, BPW)], hv.at[pl.ds(0, BPW)])
        pltpu.sync_copy(rels_hbm.at[pl.ds(base, BPW)], rv.at[pl.ds(0, BPW)])
        pltpu.sync_copy(tails_hbm.at[pl.ds(base, BPW)], tv.at[pl.ds(0, BPW)])
        bufs = (b0, b1, b2, b3, b4, b5)
        outs = (hh_out, ht_out, th_out, tt_out, r_out, rinv_out)
        for c in range(NCHUNK):
            cbase = c * CHUNK

            @pl.loop(0, CHUNK)
            def _(i):
                h = hv[pl.ds(cbase + i, 16)][0]
                t = tv[pl.ds(cbase + i, 16)][0]
                r = rv[pl.ds(cbase + i, 16)][0]
                dst = pl.ds(i, 1)
                pltpu.async_copy(
                    enth_hbm.at[pl.ds(h, 1)], b0.at[dst], sem)
                pltpu.async_copy(
                    enth_hbm.at[pl.ds(t, 1)], b1.at[dst], sem)
                pltpu.async_copy(
                    entt_hbm.at[pl.ds(h, 1)], b2.at[dst], sem)
                pltpu.async_copy(
                    entt_hbm.at[pl.ds(t, 1)], b3.at[dst], sem)
                pltpu.async_copy(
                    rel_hbm.at[pl.ds(r, 1)], b4.at[dst], sem)
                pltpu.async_copy(
                    relinv_hbm.at[pl.ds(r, 1)], b5.at[dst], sem)

            for buf in bufs:
                pltpu.make_async_copy(
                    enth_hbm.at[pl.ds(0, CHUNK)], buf, sem).wait()
            for buf, out in zip(bufs, outs):
                @pl.loop(0, CHUNK)
                def _(q, _buf=buf):
                    for kk in range(D // 16):
                        slc = (pl.ds(q, 1), pl.ds(kk * 16, 16))
                        pk.at[slc][...] = _buf.at[slc][...]

                pltpu.sync_copy(pk, out.at[pl.ds(base + cbase, CHUNK)])

    return k(heads, rels, tails, ent_h, ent_t, rel, rel_inv)


def _tc_score(hh, ht, th, tt, r, rinv):
    blk = 2048

    def body(hh_ref, ht_ref, th_ref, tt_ref, r_ref, rinv_ref, o_ref):
        s = (slice(None), slice(0, D))
        f = jnp.sum(hh_ref[s] * r_ref[s] * tt_ref[s], axis=1)
        inv = jnp.sum(ht_ref[s] * rinv_ref[s] * th_ref[s], axis=1)
        o_ref[...] = jnp.clip((f + inv) * 0.5, -20.0, 20.0)

    return pl.pallas_call(
        body,
        out_shape=jax.ShapeDtypeStruct((BATCH,), jnp.float32),
        grid=(BATCH // blk,),
        in_specs=[pl.BlockSpec((blk, DP), lambda i: (i, 0))] * 6,
        out_specs=pl.BlockSpec((blk,), lambda i: (i,)),
    )(hh, ht, th, tt, r, rinv)


def kernel(heads, rels, tails, ent_h_embs, ent_t_embs, rel_embs, rel_inv_embs):
    heads = heads.astype(jnp.int32)
    rels = rels.astype(jnp.int32)
    tails = tails.astype(jnp.int32)
    hh, ht, th, tt, r, rinv = _sc_gather_all(
        heads, rels, tails, ent_h_embs, ent_t_embs, rel_embs, rel_inv_embs)
    return _tc_score(hh, ht, th, tt, r, rinv)
